# baseline (speedup 1.0000x reference)
import functools

import jax
import jax.numpy as jnp
from jax import lax
from jax.experimental import pallas as pl
from jax.experimental.pallas import tpu as pltpu
from jax.experimental.pallas import tpu_sc as plsc

N_NODES = 10000
NP = 10240
E = 320000
D = 128
G = 64
NC = 2
NS = 16
NW = NC * NS
EPW = E // NW
HALF = 5120
AROWS = 5248
RPS = AROWS // NS
EBLK = 2000
CB = 256
QCH = 41
QCAP = QCH * CB
SENT_SRC = N_NODES + 200
BLK = 1024
NBLK = NP // BLK

_mesh = plsc.VectorSubcoreMesh(core_axis_name="c", subcore_axis_name="s")


@functools.partial(
    pl.kernel,
    out_type=[
        jax.ShapeDtypeStruct((NC, NW, QCAP), jnp.int32),
        jax.ShapeDtypeStruct((NC, NW, QCAP), jnp.int32),
        jax.ShapeDtypeStruct((NW, 16), jnp.int32),
        jax.ShapeDtypeStruct((NW, NP // 16, 16), jnp.float32),
    ],
    mesh=_mesh,
    scratch_types=[
        pltpu.VMEM((EBLK,), jnp.int32),
        pltpu.VMEM((EBLK,), jnp.int32),
        pltpu.VMEM((NP // 16, 16), jnp.float32),
        pltpu.VMEM((QCAP,), jnp.int32),
        pltpu.VMEM((QCAP,), jnp.int32),
        pltpu.VMEM((QCAP,), jnp.int32),
        pltpu.VMEM((QCAP,), jnp.int32),
        pltpu.VMEM((16,), jnp.int32),
    ],
    compiler_params=pltpu.CompilerParams(needs_layout_passes=False),
)
def _sc_partition(src_hbm, dst_hbm, qsrc_hbm, qdst_hbm, cnt_hbm, deg_hbm,
                  sidx, didx, hist, q0s, q0d, q1s, q1d, cvec):
    cid = lax.axis_index("c")
    sid = lax.axis_index("s")
    wid = cid * NS + sid

    zeros16 = jnp.zeros((16,), jnp.float32)

    def zinit(i, carry):
        hist[i, :] = zeros16
        return carry

    lax.fori_loop(0, NP // 16, zinit, 0)

    ones16 = jnp.ones((16,), jnp.float32)

    def body(ii, offs):
        off0, off1 = offs
        dvec = didx[pl.ds(ii * 16, 16)]
        svec = sidx[pl.ds(ii * 16, 16)]
        plsc.addupdate_scatter(hist, [dvec >> 4, dvec & 15], ones16)
        m0 = dvec < HALF
        plsc.store_compressed(q0d.at[pl.ds(off0, 16)], dvec, mask=m0)
        plsc.store_compressed(q0s.at[pl.ds(off0, 16)], svec, mask=m0)
        m1 = jnp.logical_not(m0)
        plsc.store_compressed(q1d.at[pl.ds(off1, 16)], dvec - HALF, mask=m1)
        plsc.store_compressed(q1s.at[pl.ds(off1, 16)], svec, mask=m1)
        c0 = jnp.sum(m0.astype(jnp.int32))
        return (off0 + c0, off1 + 16 - c0)

    def blk(b, offs):
        pltpu.sync_copy(src_hbm.at[wid].at[b], sidx)
        pltpu.sync_copy(dst_hbm.at[wid].at[b], didx)
        return lax.fori_loop(0, EBLK // 16, body, offs)

    off0, off1 = lax.fori_loop(
        0, EPW // EBLK, blk, (jnp.int32(0), jnp.int32(0))
    )

    sent_d = jnp.full((16,), HALF, jnp.int32)
    sent_s = jnp.full((16,), SENT_SRC, jnp.int32)
    for k in range(CB // 16):
        q0d[pl.ds(off0 + k * 16, 16)] = sent_d
        q0s[pl.ds(off0 + k * 16, 16)] = sent_s
        q1d[pl.ds(off1 + k * 16, 16)] = sent_d
        q1s[pl.ds(off1 + k * 16, 16)] = sent_s

    lane = lax.broadcasted_iota(jnp.int32, (16,), 0)
    cvec[...] = jnp.where(lane == 0, off0, jnp.where(lane == 1, off1, 0))
    pltpu.sync_copy(cvec, cnt_hbm.at[wid])
    pltpu.sync_copy(q0s, qsrc_hbm.at[0].at[wid])
    pltpu.sync_copy(q0d, qdst_hbm.at[0].at[wid])
    pltpu.sync_copy(q1s, qsrc_hbm.at[1].at[wid])
    pltpu.sync_copy(q1d, qdst_hbm.at[1].at[wid])
    pltpu.sync_copy(hist, deg_hbm.at[wid])


@functools.partial(
    pl.kernel,
    out_type=jax.ShapeDtypeStruct((NC, AROWS, D), jnp.float32),
    mesh=_mesh,
    scratch_types=[
        pltpu.VMEM((QCAP,), jnp.int32),
        pltpu.VMEM((QCAP,), jnp.int32),
        pltpu.VMEM((16,), jnp.int32),
        pltpu.VMEM((CB, D), jnp.float32),
        pltpu.VMEM_SHARED((AROWS, D), jnp.float32),
    ],
    compiler_params=pltpu.CompilerParams(needs_layout_passes=False),
)
def _sc_edge_agg(hs_hbm, qsrc_hbm, qdst_hbm, cnt_hbm, zeros_hbm, out_hbm,
                 qs, qd, cv, rows, acc):
    cid = lax.axis_index("c")
    sid = lax.axis_index("s")
    pltpu.sync_copy(zeros_hbm.at[pl.ds(sid * RPS, RPS)], acc.at[pl.ds(sid * RPS, RPS)])
    plsc.subcore_barrier()
    lane = lax.broadcasted_iota(jnp.int32, (16,), 0)

    for k in range(2):
        w = 2 * sid + k
        pltpu.sync_copy(qsrc_hbm.at[cid].at[w], qs)
        pltpu.sync_copy(qdst_hbm.at[cid].at[w], qd)
        pltpu.sync_copy(cnt_hbm.at[w], cv)
        n = jnp.sum(jnp.where(lane == cid, cv[...], 0))
        nch = (n + CB - 1) // CB

        def body(j, carry):
            pltpu.sync_copy(hs_hbm.at[qs.at[pl.ds(j * CB, CB)]], rows)
            pltpu.sync_copy(rows, acc.at[qd.at[pl.ds(j * CB, CB)]], add=True)
            return carry

        lax.fori_loop(0, nch, body, 0)

    plsc.subcore_barrier()
    pltpu.sync_copy(acc.at[pl.ds(sid * RPS, RPS)], out_hbm.at[cid].at[pl.ds(sid * RPS, RPS)])


def _mm_body(deg_ref, x_ref, w_ref, hs_ref, dinv_ref):
    total = jnp.sum(deg_ref[...], axis=1, keepdims=True) + 1.0
    dinv = lax.rsqrt(total)
    h = jnp.dot(x_ref[...], w_ref[...], preferred_element_type=jnp.float32)
    hs_ref[...] = h * dinv
    dinv_ref[...] = dinv


def _tc_matmul(deg_t, x_pad, W1):
    return pl.pallas_call(
        _mm_body,
        grid=(NBLK,),
        in_specs=[
            pl.BlockSpec((BLK, NW), lambda i: (i, 0)),
            pl.BlockSpec((BLK, D), lambda i: (i, 0)),
            pl.BlockSpec((D, D), lambda i: (0, 0)),
        ],
        out_specs=[
            pl.BlockSpec((BLK, D), lambda i: (i, 0)),
            pl.BlockSpec((BLK, 1), lambda i: (i, 0)),
        ],
        out_shape=[
            jax.ShapeDtypeStruct((NP, D), jnp.float32),
            jax.ShapeDtypeStruct((NP, 1), jnp.float32),
        ],
    )(deg_t, x_pad, W1)


def _head_body(accr, hs, dinv, brow, b1r, fcw, fcbr, out, sums, counts):
    i = pl.program_id(0)

    @pl.when(i == 0)
    def _():
        sums[...] = jnp.zeros_like(sums)
        counts[...] = jnp.zeros_like(counts)

    r = dinv[...] * (accr[0] + hs[...]) + b1r[...]
    r = jnp.maximum(r, 0.0)
    iota = lax.broadcasted_iota(jnp.int32, (G, BLK), 0)
    oh = (jnp.broadcast_to(brow[...], (G, BLK)) == iota).astype(jnp.float32)
    sums[...] += jnp.dot(oh, r, preferred_element_type=jnp.float32)
    counts[...] += jnp.sum(oh, axis=1, keepdims=True)

    @pl.when(i == NBLK - 1)
    def _():
        g = sums[...] / jnp.maximum(counts[...], 1.0)
        logits = jnp.dot(g, fcw[...], preferred_element_type=jnp.float32) + fcbr[...]
        m = jnp.max(logits, axis=1, keepdims=True)
        lse = m + jnp.log(jnp.sum(jnp.exp(logits - m), axis=1, keepdims=True))
        out[...] = logits - lse


def _tc_head(acc, hs, dinv, batch_row, b1r, fcW, fcbr):
    return pl.pallas_call(
        _head_body,
        grid=(NBLK,),
        in_specs=[
            pl.BlockSpec((1, BLK, D), lambda i: (i // 5, i % 5, 0)),
            pl.BlockSpec((BLK, D), lambda i: (i, 0)),
            pl.BlockSpec((BLK, 1), lambda i: (i, 0)),
            pl.BlockSpec((1, BLK), lambda i: (0, i)),
            pl.BlockSpec((1, D), lambda i: (0, 0)),
            pl.BlockSpec((D, 2), lambda i: (0, 0)),
            pl.BlockSpec((1, 2), lambda i: (0, 0)),
        ],
        out_specs=pl.BlockSpec((G, 2), lambda i: (0, 0)),
        out_shape=jax.ShapeDtypeStruct((G, 2), jnp.float32),
        scratch_shapes=[
            pltpu.VMEM((G, D), jnp.float32),
            pltpu.VMEM((G, 1), jnp.float32),
        ],
        compiler_params=pltpu.CompilerParams(
            dimension_semantics=("arbitrary",),
        ),
    )(acc, hs, dinv, batch_row, b1r, fcW, fcbr)


def kernel(x, edge_index, batch, W1, b1, fcW, fcb):
    src = edge_index[0].astype(jnp.int32)
    dst = edge_index[1].astype(jnp.int32)
    src_w = src.reshape(NW, EPW // EBLK, EBLK)
    dst_w = dst.reshape(NW, EPW // EBLK, EBLK)
    x_pad = jnp.pad(x, ((0, NP - N_NODES), (0, 0)))
    batch_row = jnp.pad(
        batch.astype(jnp.int32), (0, NP - N_NODES), constant_values=G
    ).reshape(1, NP)
    zeros_acc = jnp.zeros((AROWS, D), jnp.float32)

    qsrc, qdst, counts, deg_part = _sc_partition(src_w, dst_w)
    deg_t = deg_part.reshape(NW, NP).T
    hs, dinv = _tc_matmul(deg_t, x_pad, W1)
    acc = _sc_edge_agg(
        hs,
        qsrc,
        qdst,
        counts,
        zeros_acc,
    )
    out = _tc_head(
        acc, hs, dinv, batch_row,
        b1.reshape(1, D), fcW, fcb.reshape(1, 2),
    )
    return out


# speedup vs baseline: 1.9232x; 1.9232x over previous
import functools

import jax
import jax.numpy as jnp
from jax import lax
from jax.experimental import pallas as pl
from jax.experimental.pallas import tpu as pltpu
from jax.experimental.pallas import tpu_sc as plsc

N_NODES = 10000
NP = 10240
E = 320000
D = 128
G = 64
NC = 2
NS = 16
NW = NC * NS
EPW = E // NW
C = 125
NCH = EPW // C
RPS = NP // NS
BLK = 1024
NBLK = NP // BLK

_mesh = plsc.VectorSubcoreMesh(core_axis_name="c", subcore_axis_name="s")


@functools.partial(
    pl.kernel,
    out_type=jax.ShapeDtypeStruct((NW, NP // 16, 16), jnp.float32),
    mesh=_mesh,
    scratch_types=[
        pltpu.VMEM((EPW,), jnp.int32),
        pltpu.VMEM((NP // 16, 16), jnp.float32),
    ],
    compiler_params=pltpu.CompilerParams(needs_layout_passes=False),
)
def _sc_degree(dst_hbm, out_hbm, didx, hist):
    cid = lax.axis_index("c")
    sid = lax.axis_index("s")
    wid = cid * NS + sid
    pltpu.sync_copy(dst_hbm.at[wid], didx)

    zeros = jnp.zeros((16,), jnp.float32)

    def zinit(i, carry):
        hist[i, :] = zeros
        return carry

    lax.fori_loop(0, NP // 16, zinit, 0)

    ones = jnp.ones((16,), jnp.float32)

    def body(i, carry):
        idx = didx[pl.ds(i * 16, 16)]
        plsc.addupdate_scatter(hist, [idx >> 4, idx & 15], ones)
        return carry

    lax.fori_loop(0, EPW // 16, body, 0)
    pltpu.sync_copy(hist, out_hbm.at[wid])


@functools.partial(
    pl.kernel,
    out_type=jax.ShapeDtypeStruct((NC, NP, D), jnp.float32),
    mesh=_mesh,
    scratch_types=[
        pltpu.VMEM((NCH, C), jnp.int32),
        pltpu.VMEM((NCH, C), jnp.int32),
        pltpu.VMEM((C, D), jnp.float32),
        pltpu.VMEM((C, D), jnp.float32),
        pltpu.VMEM_SHARED((NP, D), jnp.float32),
    ],
    compiler_params=pltpu.CompilerParams(needs_layout_passes=False),
)
def _sc_edge_agg(hs_hbm, src_hbm, dst_hbm, zeros_hbm, out_hbm,
                 sidx, didx, rows0, rows1, acc):
    cid = lax.axis_index("c")
    sid = lax.axis_index("s")
    wid = cid * NS + sid
    pltpu.sync_copy(src_hbm.at[wid], sidx)
    pltpu.sync_copy(dst_hbm.at[wid], didx)
    pltpu.sync_copy(zeros_hbm.at[pl.ds(sid * RPS, RPS)], acc.at[pl.ds(sid * RPS, RPS)])
    plsc.subcore_barrier()

    def body(j, carry):
        pltpu.sync_copy(hs_hbm.at[sidx.at[j]], rows0)
        pltpu.sync_copy(rows0, acc.at[didx.at[j]], add=True)
        return carry

    lax.fori_loop(0, NCH, body, 0)
    plsc.subcore_barrier()
    pltpu.sync_copy(acc.at[pl.ds(sid * RPS, RPS)], out_hbm.at[cid].at[pl.ds(sid * RPS, RPS)])


def _mm_body(deg_ref, x_ref, w_ref, hs_ref, dinv_ref):
    total = jnp.sum(deg_ref[...], axis=1, keepdims=True) + 1.0
    dinv = lax.rsqrt(total)
    h = jnp.dot(x_ref[...], w_ref[...], preferred_element_type=jnp.float32)
    hs_ref[...] = h * dinv
    dinv_ref[...] = dinv


def _tc_matmul(deg_t, x_pad, W1):
    return pl.pallas_call(
        _mm_body,
        grid=(NBLK,),
        in_specs=[
            pl.BlockSpec((BLK, NW), lambda i: (i, 0)),
            pl.BlockSpec((BLK, D), lambda i: (i, 0)),
            pl.BlockSpec((D, D), lambda i: (0, 0)),
        ],
        out_specs=[
            pl.BlockSpec((BLK, D), lambda i: (i, 0)),
            pl.BlockSpec((BLK, 1), lambda i: (i, 0)),
        ],
        out_shape=[
            jax.ShapeDtypeStruct((NP, D), jnp.float32),
            jax.ShapeDtypeStruct((NP, 1), jnp.float32),
        ],
    )(deg_t, x_pad, W1)


def _head_body(accA, accB, hs, dinv, brow, b1r, fcw, fcbr, out, sums, counts):
    i = pl.program_id(0)

    @pl.when(i == 0)
    def _():
        sums[...] = jnp.zeros_like(sums)
        counts[...] = jnp.zeros_like(counts)

    r = dinv[...] * (accA[...] + accB[...] + hs[...]) + b1r[...]
    r = jnp.maximum(r, 0.0)
    iota = lax.broadcasted_iota(jnp.int32, (G, BLK), 0)
    oh = (jnp.broadcast_to(brow[...], (G, BLK)) == iota).astype(jnp.float32)
    sums[...] += jnp.dot(oh, r, preferred_element_type=jnp.float32)
    counts[...] += jnp.sum(oh, axis=1, keepdims=True)

    @pl.when(i == NBLK - 1)
    def _():
        g = sums[...] / jnp.maximum(counts[...], 1.0)
        logits = jnp.dot(g, fcw[...], preferred_element_type=jnp.float32) + fcbr[...]
        m = jnp.max(logits, axis=1, keepdims=True)
        lse = m + jnp.log(jnp.sum(jnp.exp(logits - m), axis=1, keepdims=True))
        out[...] = logits - lse


def _tc_head(accA, accB, hs, dinv, batch_row, b1r, fcW, fcbr):
    return pl.pallas_call(
        _head_body,
        grid=(NBLK,),
        in_specs=[
            pl.BlockSpec((BLK, D), lambda i: (i, 0)),
            pl.BlockSpec((BLK, D), lambda i: (i, 0)),
            pl.BlockSpec((BLK, D), lambda i: (i, 0)),
            pl.BlockSpec((BLK, 1), lambda i: (i, 0)),
            pl.BlockSpec((1, BLK), lambda i: (0, i)),
            pl.BlockSpec((1, D), lambda i: (0, 0)),
            pl.BlockSpec((D, 2), lambda i: (0, 0)),
            pl.BlockSpec((1, 2), lambda i: (0, 0)),
        ],
        out_specs=pl.BlockSpec((G, 2), lambda i: (0, 0)),
        out_shape=jax.ShapeDtypeStruct((G, 2), jnp.float32),
        scratch_shapes=[
            pltpu.VMEM((G, D), jnp.float32),
            pltpu.VMEM((G, 1), jnp.float32),
        ],
        compiler_params=pltpu.CompilerParams(
            dimension_semantics=("arbitrary",),
        ),
    )(accA, accB, hs, dinv, batch_row, b1r, fcW, fcbr)


def kernel(x, edge_index, batch, W1, b1, fcW, fcb):
    src = edge_index[0].astype(jnp.int32)
    dst = edge_index[1].astype(jnp.int32)
    dst_w = dst.reshape(NW, EPW)
    src_ch = src.reshape(NW, NCH, C)
    dst_ch = dst.reshape(NW, NCH, C)
    x_pad = jnp.pad(x, ((0, NP - N_NODES), (0, 0)))
    batch_row = jnp.pad(
        batch.astype(jnp.int32), (0, NP - N_NODES), constant_values=G
    ).reshape(1, NP)
    zeros_nd = jnp.zeros((NP, D), jnp.float32)

    deg_part = _sc_degree(dst_w)
    deg_t = deg_part.reshape(NW, NP).T
    hs, dinv = _tc_matmul(deg_t, x_pad, W1)
    acc = _sc_edge_agg(hs, src_ch, dst_ch, zeros_nd)
    out = _tc_head(
        acc[0], acc[1], hs, dinv, batch_row,
        b1.reshape(1, D), fcW, fcb.reshape(1, 2),
    )
    return out
